# baseline (speedup 1.0000x reference)
import functools

import jax
import jax.numpy as jnp
from jax import lax
from jax.experimental import pallas as pl
from jax.experimental.pallas import tpu as pltpu
from jax.experimental.pallas import tpu_sc as plsc

VOCAB = 100000
D = 64
T = 200
B = 4096
NC = 2
NS = 16
NW = NC * NS
BB = B // NW
LANES = 16
NSTEP = T // 2


def _emb_body(ids_hbm, tok_hbm, pos_hbm, out_hbm,
              idx0, idx1, rows0, rows1, tr0, tr1, pos_v,
              gsem0, gsem1, wsem0, wsem1):
    wid = lax.axis_index("s") * NC + lax.axis_index("c")
    bcol = wid * BB
    pltpu.sync_copy(pos_hbm, pos_v)

    idx = (idx0, idx1)
    rows = (rows0, rows1)
    tr = (tr0, tr1)
    gsem = (gsem0, gsem1)
    wsem = (wsem0, wsem1)

    iota = lax.iota(jnp.int32, LANES)
    row_idx = [iota + LANES * i for i in range(BB // LANES)]

    def transpose_add(rbuf, tbuf, t):
        t64 = t * D

        def d8_body(d8, c):
            for dd in range(8):
                d = d8 * 8 + dd
                pv = plsc.load_gather(
                    pos_v, [jnp.full((LANES,), t64 + d, jnp.int32)])
                col = jnp.full((LANES,), d, jnp.int32)
                for i in range(BB // LANES):
                    v = plsc.load_gather(rbuf, [row_idx[i], col])
                    tbuf[d, pl.ds(LANES * i, LANES)] = v + pv
            return c

        lax.fori_loop(0, D // 8, d8_body, 0)

    pltpu.sync_copy(ids_hbm.at[0, pl.ds(bcol, BB)], idx0)
    pltpu.async_copy(tok_hbm.at[idx0], rows0, gsem0)

    def step_body(s, carry):
        for b in range(2):
            t = 2 * s + b
            nb = 1 - b
            if b == 0:
                pltpu.sync_copy(ids_hbm.at[t + 1, pl.ds(bcol, BB)], idx[nb])

                @pl.when(s > 0)
                def _wait_prev_write():
                    pltpu.make_async_copy(
                        tr[nb], out_hbm.at[t - 1, :, pl.ds(bcol, BB)], wsem[nb]
                    ).wait()

                pltpu.async_copy(tok_hbm.at[idx[nb]], rows[nb], gsem[nb])
            else:
                @pl.when(s < NSTEP - 1)
                def _stage_next():
                    pltpu.sync_copy(ids_hbm.at[t + 1, pl.ds(bcol, BB)], idx[nb])
                    pltpu.make_async_copy(
                        tr[nb], out_hbm.at[t - 1, :, pl.ds(bcol, BB)], wsem[nb]
                    ).wait()
                    pltpu.async_copy(tok_hbm.at[idx[nb]], rows[nb], gsem[nb])

            pltpu.make_async_copy(tok_hbm.at[idx[b]], rows[b], gsem[b]).wait()
            transpose_add(rows[b], tr[b], t)
            pltpu.async_copy(tr[b], out_hbm.at[t, :, pl.ds(bcol, BB)], wsem[b])
        return carry

    lax.fori_loop(0, NSTEP, step_body, 0)

    pltpu.make_async_copy(
        tr0, out_hbm.at[T - 2, :, pl.ds(bcol, BB)], wsem0
    ).wait()
    pltpu.make_async_copy(
        tr1, out_hbm.at[T - 1, :, pl.ds(bcol, BB)], wsem1
    ).wait()


_emb_kernel = functools.partial(
    pl.kernel,
    out_type=jax.ShapeDtypeStruct((T, D, B), jnp.float32),
    mesh=plsc.VectorSubcoreMesh(core_axis_name="c", subcore_axis_name="s"),
    scratch_types=[
        pltpu.VMEM((BB,), jnp.int32),
        pltpu.VMEM((BB,), jnp.int32),
        pltpu.VMEM((BB, D), jnp.float32),
        pltpu.VMEM((BB, D), jnp.float32),
        pltpu.VMEM((D, BB), jnp.float32),
        pltpu.VMEM((D, BB), jnp.float32),
        pltpu.VMEM((D * T,), jnp.float32),
        pltpu.SemaphoreType.DMA,
        pltpu.SemaphoreType.DMA,
        pltpu.SemaphoreType.DMA,
        pltpu.SemaphoreType.DMA,
    ],
    compiler_params=pltpu.CompilerParams(
        use_tc_tiling_on_sc=False, needs_layout_passes=False),
)(_emb_body)


def kernel(input_ids, tok_table, pos_table):
    batch, block = input_ids.shape
    ids_t = input_ids.T.astype(jnp.int32)
    pos_t = pos_table.reshape(-1)
    out_t = _emb_kernel(ids_t, tok_table, pos_t)
    return jnp.transpose(out_t, (2, 0, 1))


# speedup vs baseline: 1.8287x; 1.8287x over previous
import functools

import jax
import jax.numpy as jnp
from jax import lax
from jax.experimental import pallas as pl
from jax.experimental.pallas import tpu as pltpu
from jax.experimental.pallas import tpu_sc as plsc

VOCAB = 100000
D = 64
T = 200
B = 4096
NC = 2
NS = 16
NW = NC * NS
BB = B // NW
LANES = 16
NSTEP = T // 2


def _emb_body(ids_hbm, tok_hbm, pos_hbm, out_hbm,
              idx0, idx1, rows0, rows1, tr0, tr1, pos_v,
              gsem0, gsem1, wsem0, wsem1):
    wid = lax.axis_index("s") * NC + lax.axis_index("c")
    bcol = wid * BB
    pltpu.sync_copy(pos_hbm, pos_v)

    idx = (idx0, idx1)
    rows = (rows0, rows1)
    tr = (tr0, tr1)
    gsem = (gsem0, gsem1)
    wsem = (wsem0, wsem1)

    iota = lax.iota(jnp.int32, LANES)
    rvec = [iota + LANES * i for i in range(BB // LANES)]
    pbase = [(iota + s) & 15 for s in range(LANES)]

    def transpose_add(rbuf, tbuf, t):
        t64 = t * D

        def m_body(m, c):
            m16 = m * LANES
            cvec = [pbase[s] + m16 for s in range(LANES)]
            pv = [plsc.load_gather(pos_v, [cvec[s] + t64])
                  for s in range(LANES)]
            for i in range(BB // LANES):
                for s in range(LANES):
                    v = plsc.load_gather(rbuf, [rvec[i], cvec[s]])
                    plsc.store_scatter(tbuf, [cvec[s], rvec[i]], v + pv[s])
            return c

        lax.fori_loop(0, D // LANES, m_body, 0)

    pltpu.sync_copy(ids_hbm.at[0, pl.ds(bcol, BB)], idx0)
    pltpu.async_copy(tok_hbm.at[idx0], rows0, gsem0)

    def step_body(s, carry):
        for b in range(2):
            t = 2 * s + b
            nb = 1 - b
            if b == 0:
                pltpu.sync_copy(ids_hbm.at[t + 1, pl.ds(bcol, BB)], idx[nb])

                @pl.when(s > 0)
                def _wait_prev_write():
                    pltpu.make_async_copy(
                        tr[nb], out_hbm.at[t - 1, :, pl.ds(bcol, BB)], wsem[nb]
                    ).wait()

                pltpu.async_copy(tok_hbm.at[idx[nb]], rows[nb], gsem[nb])
            else:
                @pl.when(s < NSTEP - 1)
                def _stage_next():
                    pltpu.sync_copy(ids_hbm.at[t + 1, pl.ds(bcol, BB)], idx[nb])
                    pltpu.make_async_copy(
                        tr[nb], out_hbm.at[t - 1, :, pl.ds(bcol, BB)], wsem[nb]
                    ).wait()
                    pltpu.async_copy(tok_hbm.at[idx[nb]], rows[nb], gsem[nb])

            pltpu.make_async_copy(tok_hbm.at[idx[b]], rows[b], gsem[b]).wait()
            transpose_add(rows[b], tr[b], t)
            pltpu.async_copy(tr[b], out_hbm.at[t, :, pl.ds(bcol, BB)], wsem[b])
        return carry

    lax.fori_loop(0, NSTEP, step_body, 0)

    pltpu.make_async_copy(
        tr0, out_hbm.at[T - 2, :, pl.ds(bcol, BB)], wsem0
    ).wait()
    pltpu.make_async_copy(
        tr1, out_hbm.at[T - 1, :, pl.ds(bcol, BB)], wsem1
    ).wait()


_emb_kernel = functools.partial(
    pl.kernel,
    out_type=jax.ShapeDtypeStruct((T, D, B), jnp.float32),
    mesh=plsc.VectorSubcoreMesh(core_axis_name="c", subcore_axis_name="s"),
    scratch_types=[
        pltpu.VMEM((BB,), jnp.int32),
        pltpu.VMEM((BB,), jnp.int32),
        pltpu.VMEM((BB, D), jnp.float32),
        pltpu.VMEM((BB, D), jnp.float32),
        pltpu.VMEM((D, BB), jnp.float32),
        pltpu.VMEM((D, BB), jnp.float32),
        pltpu.VMEM((D * T,), jnp.float32),
        pltpu.SemaphoreType.DMA,
        pltpu.SemaphoreType.DMA,
        pltpu.SemaphoreType.DMA,
        pltpu.SemaphoreType.DMA,
    ],
    compiler_params=pltpu.CompilerParams(
        use_tc_tiling_on_sc=False, needs_layout_passes=False),
)(_emb_body)


def kernel(input_ids, tok_table, pos_table):
    batch, block = input_ids.shape
    ids_t = input_ids.T.astype(jnp.int32)
    pos_t = pos_table.reshape(-1)
    out_t = _emb_kernel(ids_t, tok_table, pos_t)
    return jnp.transpose(out_t, (2, 0, 1))


# speedup vs baseline: 1.9395x; 1.0606x over previous
import functools

import jax
import jax.numpy as jnp
from jax import lax
from jax.experimental import pallas as pl
from jax.experimental.pallas import tpu as pltpu
from jax.experimental.pallas import tpu_sc as plsc

VOCAB = 100000
D = 64
T = 200
B = 4096
NC = 2
NS = 16
NW = NC * NS
BB = B // NW
LANES = 16
NSTEP = T // 2


def _emb_body(ids_hbm, tok_hbm, pos_hbm, out_hbm,
              idx0, idx1, rows0, rows1, tr0, tr1, pos_v,
              gsem0, gsem1, wsem0, wsem1):
    wid = lax.axis_index("s") * NC + lax.axis_index("c")
    bcol = wid * BB
    pltpu.sync_copy(pos_hbm, pos_v)

    idx = (idx0, idx1)
    rows = (rows0, rows1)
    tr = (tr0, tr1)
    gsem = (gsem0, gsem1)
    wsem = (wsem0, wsem1)

    iota = lax.iota(jnp.int32, LANES)
    rvec = [iota + LANES * i for i in range(BB // LANES)]
    pbase = [(iota + s) & 15 for s in range(LANES)]

    def transpose_add(rbuf, tbuf, t):
        t64 = t * D
        HALF = LANES // 2

        @plsc.parallel_loop(0, D // LANES)
        def m_body(m):
            m16 = m * LANES
            cvec = [pbase[s] + m16 for s in range(LANES)]
            pv = [plsc.load_gather(pos_v, [cvec[s] + t64])
                  for s in range(LANES)]
            for i in range(BB // LANES):
                for h in range(LANES // HALF):
                    s0 = h * HALF
                    vs = [plsc.load_gather(rbuf, [rvec[i], cvec[s0 + k]])
                          for k in range(HALF)]
                    vs = [vs[k] + pv[s0 + k] for k in range(HALF)]
                    for k in range(HALF):
                        plsc.store_scatter(
                            tbuf, [cvec[s0 + k], rvec[i]], vs[k])

    pltpu.sync_copy(ids_hbm.at[0, pl.ds(bcol, BB)], idx0)
    pltpu.async_copy(tok_hbm.at[idx0], rows0, gsem0)

    def step_body(s, carry):
        for b in range(2):
            t = 2 * s + b
            nb = 1 - b
            if b == 0:
                pltpu.sync_copy(ids_hbm.at[t + 1, pl.ds(bcol, BB)], idx[nb])

                @pl.when(s > 0)
                def _wait_prev_write():
                    pltpu.make_async_copy(
                        tr[nb], out_hbm.at[t - 1, :, pl.ds(bcol, BB)], wsem[nb]
                    ).wait()

                pltpu.async_copy(tok_hbm.at[idx[nb]], rows[nb], gsem[nb])
            else:
                @pl.when(s < NSTEP - 1)
                def _stage_next():
                    pltpu.sync_copy(ids_hbm.at[t + 1, pl.ds(bcol, BB)], idx[nb])
                    pltpu.make_async_copy(
                        tr[nb], out_hbm.at[t - 1, :, pl.ds(bcol, BB)], wsem[nb]
                    ).wait()
                    pltpu.async_copy(tok_hbm.at[idx[nb]], rows[nb], gsem[nb])

            pltpu.make_async_copy(tok_hbm.at[idx[b]], rows[b], gsem[b]).wait()
            transpose_add(rows[b], tr[b], t)
            pltpu.async_copy(tr[b], out_hbm.at[t, :, pl.ds(bcol, BB)], wsem[b])
        return carry

    lax.fori_loop(0, NSTEP, step_body, 0)

    pltpu.make_async_copy(
        tr0, out_hbm.at[T - 2, :, pl.ds(bcol, BB)], wsem0
    ).wait()
    pltpu.make_async_copy(
        tr1, out_hbm.at[T - 1, :, pl.ds(bcol, BB)], wsem1
    ).wait()


_emb_kernel = functools.partial(
    pl.kernel,
    out_type=jax.ShapeDtypeStruct((T, D, B), jnp.float32),
    mesh=plsc.VectorSubcoreMesh(core_axis_name="c", subcore_axis_name="s"),
    scratch_types=[
        pltpu.VMEM((BB,), jnp.int32),
        pltpu.VMEM((BB,), jnp.int32),
        pltpu.VMEM((BB, D), jnp.float32),
        pltpu.VMEM((BB, D), jnp.float32),
        pltpu.VMEM((D, BB), jnp.float32),
        pltpu.VMEM((D, BB), jnp.float32),
        pltpu.VMEM((D * T,), jnp.float32),
        pltpu.SemaphoreType.DMA,
        pltpu.SemaphoreType.DMA,
        pltpu.SemaphoreType.DMA,
        pltpu.SemaphoreType.DMA,
    ],
    compiler_params=pltpu.CompilerParams(
        use_tc_tiling_on_sc=False, needs_layout_passes=False),
)(_emb_body)


def kernel(input_ids, tok_table, pos_table):
    batch, block = input_ids.shape
    ids_t = input_ids.T.astype(jnp.int32)
    pos_t = pos_table.reshape(-1)
    out_t = _emb_kernel(ids_t, tok_table, pos_t)
    return jnp.transpose(out_t, (2, 0, 1))


# speedup vs baseline: 2.4482x; 1.2623x over previous
import functools

import jax
import jax.numpy as jnp
from jax import lax
from jax.experimental import pallas as pl
from jax.experimental.pallas import tpu as pltpu
from jax.experimental.pallas import tpu_sc as plsc

VOCAB = 100000
D = 64
T = 200
B = 4096
NC = 2
NS = 16
NW = NC * NS
ROWS_PER_W = B // NW
LANES = 16
C = 4
CT = C * T
NCHUNK = ROWS_PER_W // C
NSTEP = NCHUNK // 2


def _emb_body(ids_hbm, tok_hbm, pos_hbm, out_hbm,
              idx0, idx1, rows0, rows1, pos_v,
              gsem0, gsem1, wsem0, wsem1):
    wid = lax.axis_index("s") * NC + lax.axis_index("c")
    tok_base = wid * ROWS_PER_W * T
    pltpu.sync_copy(pos_hbm, pos_v)

    idx = (idx0, idx1)
    rows = (rows0, rows1)
    gsem = (gsem0, gsem1)
    wsem = (wsem0, wsem1)

    def add_pos(rbuf):
        def add_t(t, c):
            for j in range(D // LANES):
                sl = pl.ds(j * LANES, LANES)
                pv = pos_v[t, sl]
                for cc in range(C):
                    plsc.addupdate(rbuf.at[cc * T + t, sl], pv)
            return c
        lax.fori_loop(0, T, add_t, 0)

    pltpu.sync_copy(ids_hbm.at[pl.ds(tok_base, CT)], idx0)
    pltpu.async_copy(tok_hbm.at[idx0], rows0, gsem0)

    def step_body(s, carry):
        for b in range(2):
            g = 2 * s + b
            nb = 1 - b
            tok0 = tok_base + g * CT
            if b == 0:
                pltpu.sync_copy(ids_hbm.at[pl.ds(tok0 + CT, CT)], idx[nb])

                @pl.when(s > 0)
                def _wait_prev_write():
                    pltpu.make_async_copy(
                        rows[nb], out_hbm.at[pl.ds(tok0 - CT, CT)], wsem[nb]
                    ).wait()

                pltpu.async_copy(tok_hbm.at[idx[nb]], rows[nb], gsem[nb])
            else:
                @pl.when(s < NSTEP - 1)
                def _stage_next():
                    pltpu.sync_copy(ids_hbm.at[pl.ds(tok0 + CT, CT)], idx[nb])
                    pltpu.make_async_copy(
                        rows[nb], out_hbm.at[pl.ds(tok0 - CT, CT)], wsem[nb]
                    ).wait()
                    pltpu.async_copy(tok_hbm.at[idx[nb]], rows[nb], gsem[nb])

            pltpu.make_async_copy(tok_hbm.at[idx[b]], rows[b], gsem[b]).wait()
            add_pos(rows[b])
            pltpu.async_copy(rows[b], out_hbm.at[pl.ds(tok0, CT)], wsem[b])
        return carry

    lax.fori_loop(0, NSTEP, step_body, 0)

    pltpu.make_async_copy(
        rows0, out_hbm.at[pl.ds(tok_base + (NCHUNK - 2) * CT, CT)], wsem0
    ).wait()
    pltpu.make_async_copy(
        rows1, out_hbm.at[pl.ds(tok_base + (NCHUNK - 1) * CT, CT)], wsem1
    ).wait()


_emb_kernel = functools.partial(
    pl.kernel,
    out_type=jax.ShapeDtypeStruct((B * T, D), jnp.float32),
    mesh=plsc.VectorSubcoreMesh(core_axis_name="c", subcore_axis_name="s"),
    scratch_types=[
        pltpu.VMEM((CT,), jnp.int32),
        pltpu.VMEM((CT,), jnp.int32),
        pltpu.VMEM((CT, D), jnp.float32),
        pltpu.VMEM((CT, D), jnp.float32),
        pltpu.VMEM((T, D), jnp.float32),
        pltpu.SemaphoreType.DMA,
        pltpu.SemaphoreType.DMA,
        pltpu.SemaphoreType.DMA,
        pltpu.SemaphoreType.DMA,
    ],
    compiler_params=pltpu.CompilerParams(use_tc_tiling_on_sc=False),
)(_emb_body)


def kernel(input_ids, tok_table, pos_table):
    batch, block = input_ids.shape
    ids_flat = input_ids.reshape(-1).astype(jnp.int32)
    out = _emb_kernel(ids_flat, tok_table, pos_table)
    return out.reshape(batch, block, D)
